# Initial kernel scaffold; baseline (speedup 1.0000x reference)
import jax
import jax.numpy as jnp
from jax.experimental import pallas as pl


def kernel(x, duty_cycles):
    raise NotImplementedError("write your pallas kernel here")


# speedup vs baseline: 14.4360x; 14.4360x over previous
import jax
import jax.numpy as jnp
from jax.experimental import pallas as pl
from jax.experimental.pallas import tpu as pltpu

_C = 768
_K = 77
_L = 512
_INT_MIN = -2147483648


def _kw_block(dc_ref, x_ref, o_ref):
    xb = x_ref[0]
    scale = jnp.exp(-dc_ref[...])
    boosted = xb * scale

    s = jax.lax.bitcast_convert_type(boosted, jnp.int32)
    skey = jnp.where(s < 0, s ^ jnp.int32(0x7FFFFFFF), s)

    def count_ge(cand):
        return jnp.sum((skey >= cand).astype(jnp.int32), axis=0, keepdims=True)

    zero = jnp.zeros((1, xb.shape[1]), jnp.int32)
    p = jnp.where(count_ge(zero) >= _K, zero, jnp.full_like(zero, jnp.int32(_INT_MIN)))
    for bit in range(30, -1, -1):
        cand = p | jnp.int32(1 << bit)
        p = jnp.where(count_ge(cand) >= _K, cand, p)

    s_t = jnp.where(p < 0, p ^ jnp.int32(0x7FFFFFFF), p)
    thresh = jax.lax.bitcast_convert_type(s_t, jnp.float32)
    o_ref[0] = jnp.where(boosted < thresh, jnp.zeros_like(xb), xb)


def kernel(x, duty_cycles):
    B, C, H, W = x.shape
    hw = H * W
    x3 = x.reshape(B, C, hw)
    dc = duty_cycles.reshape(C, 1)
    out = pl.pallas_call(
        _kw_block,
        grid=(B, hw // _L),
        in_specs=[
            pl.BlockSpec((C, 1), lambda b, j: (0, 0)),
            pl.BlockSpec((1, C, _L), lambda b, j: (b, 0, j)),
        ],
        out_specs=pl.BlockSpec((1, C, _L), lambda b, j: (b, 0, j)),
        out_shape=jax.ShapeDtypeStruct((B, C, hw), jnp.float32),
        compiler_params=pltpu.CompilerParams(
            dimension_semantics=("parallel", "parallel"),
        ),
    )(dc, x3)
    return out.reshape(B, C, H, W)
